# baseline (speedup 1.0000x reference)
import functools
import math

import jax
import jax.numpy as jnp
from jax import lax
from jax.experimental import pallas as pl
from jax.experimental.pallas import tpu as pltpu
from jax.experimental.pallas import tpu_sc as plsc

B = 16384
T = 50
D_MODEL = 64
LANES = 16
NUM_CORES = 2
NUM_SUBCORES = 16
NUM_WORKERS = NUM_CORES * NUM_SUBCORES
B_PER_W = B // NUM_WORKERS
SCALE = math.sqrt(D_MODEL)

_MESH = plsc.VectorSubcoreMesh(
    core_axis_name="c", subcore_axis_name="s",
    num_cores=NUM_CORES, num_subcores=NUM_SUBCORES,
)


def _gather_scale(xf, lut):
    nbuf = 3
    cw = B_PER_W
    n_chunks = T

    @functools.partial(
        pl.kernel,
        out_type=jax.ShapeDtypeStruct((B * T, D_MODEL), jnp.float32),
        mesh=_MESH,
        scratch_types=[
            pltpu.VMEM((B_PER_W * T,), jnp.int32),
            pltpu.VMEM((nbuf * cw,), jnp.int32),
            pltpu.VMEM((nbuf * cw, D_MODEL), jnp.float32),
            [pltpu.SemaphoreType.DMA] * nbuf,
            [pltpu.SemaphoreType.DMA] * nbuf,
        ],
        compiler_params=pltpu.CompilerParams(
            needs_layout_passes=False, use_tc_tiling_on_sc=False),
    )
    def body(x_hbm, lut_hbm, out_hbm, xblk_v, sidx_v, rows_v, sem_g, sem_o):
        wid = lax.axis_index("s") * NUM_CORES + lax.axis_index("c")
        b0w = pl.multiple_of(wid * B_PER_W, B_PER_W)

        iota16 = lax.iota(jnp.int32, LANES)
        iota_t = iota16 * T

        def stage_and_fire(cur, b):
            t = cur
            for g in range(cw // LANES):
                flat = iota_t + (g * LANES * T + t)
                v = plsc.load_gather(xblk_v, [flat])
                sidx_v[pl.ds(b * cw + g * LANES, LANES)] = v
            pltpu.async_copy(
                lut_hbm.at[sidx_v.at[pl.ds(b * cw, cw)]],
                rows_v.at[pl.ds(b * cw, cw)],
                sem_g[b],
            )

        def drain_gather(b):
            pltpu.make_async_copy(
                lut_hbm.at[pl.ds(0, cw)],
                rows_v.at[pl.ds(b * cw, cw)],
                sem_g[b],
            ).wait()

        def drain_out(b):
            pltpu.make_async_copy(
                out_hbm.at[pl.ds(0, cw)],
                rows_v.at[pl.ds(b * cw, cw)],
                sem_o[b],
            ).wait()

        def scale(b):
            @plsc.parallel_loop(0, cw, unroll=8)
            def _row(i):
                for j in range(D_MODEL // LANES):
                    sl = pl.ds(j * LANES, LANES)
                    rows_v[b * cw + i, sl] = rows_v[b * cw + i, sl] * SCALE

        def out_write(cur, b):
            r0 = pl.multiple_of(cur * B + b0w, B_PER_W)
            pltpu.async_copy(
                rows_v.at[pl.ds(b * cw, cw)],
                out_hbm.at[pl.ds(r0, cw)],
                sem_o[b],
            )

        f0 = pl.multiple_of(b0w * T, B_PER_W * T)
        pltpu.sync_copy(x_hbm.at[pl.ds(f0, B_PER_W * T)], xblk_v)

        for p0 in range(nbuf - 1):
            stage_and_fire(p0, p0)

        upper = n_chunks + nbuf - 1 - ((n_chunks - 1) % nbuf)

        @pl.loop(0, upper, step=nbuf)
        def _outer(s):
            for b in range(nbuf):
                cur = s + b
                bf = (b + nbuf - 1) % nbuf

                @pl.when(jnp.logical_and(cur + nbuf - 1 < n_chunks, cur >= 1))
                def _():
                    drain_out(bf)
                    stage_and_fire(cur + nbuf - 1, bf)

                @pl.when(jnp.logical_and(cur + nbuf - 1 < n_chunks, cur < 1))
                def _():
                    stage_and_fire(cur + nbuf - 1, bf)

                @pl.when(cur < n_chunks)
                def _():
                    drain_gather(b)
                    scale(b)
                    out_write(cur, b)

        for b in range(nbuf):
            drain_out(b)

    return body(xf, lut)


def _transpose_out(flat2):
    nbuf = 3
    bw = 256
    rows_c = bw // 2
    blks = B_PER_W // bw
    n_chunks = blks * T

    @functools.partial(
        pl.kernel,
        out_type=jax.ShapeDtypeStruct((T, D_MODEL, B), jnp.float32),
        mesh=_MESH,
        scratch_types=[
            pltpu.VMEM((nbuf * rows_c, 128), jnp.float32),
            pltpu.VMEM((nbuf, D_MODEL, bw), jnp.float32),
            [pltpu.SemaphoreType.DMA] * nbuf,
            [pltpu.SemaphoreType.DMA] * nbuf,
        ],
        compiler_params=pltpu.CompilerParams(needs_layout_passes=False),
    )
    def body(in_hbm, out_hbm, slab_v, trans_v, sem_g, sem_o):
        wid = lax.axis_index("s") * NUM_CORES + lax.axis_index("c")
        b0w = pl.multiple_of(wid * B_PER_W, B_PER_W)

        iota16 = lax.iota(jnp.int32, LANES)
        halfrow = lax.shift_right_logical(iota16, 1)
        parity = (iota16 & 1) * D_MODEL

        def chunk_loc(cur):
            blk = cur // T
            t = cur - blk * T
            b0 = pl.multiple_of(b0w + blk * bw, bw)
            return t, b0

        def fire(cur, b):
            t, b0 = chunk_loc(cur)
            r0 = pl.multiple_of((t * B + b0) // 2, rows_c)
            pltpu.async_copy(
                in_hbm.at[pl.ds(r0, rows_c)],
                slab_v.at[pl.ds(b * rows_c, rows_c)],
                sem_g[b],
            )

        def drain_in(b):
            pltpu.make_async_copy(
                in_hbm.at[pl.ds(0, rows_c)],
                slab_v.at[pl.ds(b * rows_c, rows_c)],
                sem_g[b],
            ).wait()

        def drain_out(b):
            pltpu.make_async_copy(
                out_hbm.at[pl.ds(0, 1), pl.ds(0, D_MODEL), pl.ds(0, bw)],
                trans_v.at[pl.ds(b, 1)],
                sem_o[b],
            ).wait()

        def transpose(b):
            ngrp = bw // LANES
            rowvecs = [halfrow + (b * rows_c + g * 8) for g in range(ngrp)]

            @plsc.parallel_loop(0, D_MODEL, unroll=4)
            def _d(d):
                colv = parity + d
                for g in range(ngrp):
                    v = plsc.load_gather(slab_v, [rowvecs[g], colv])
                    trans_v[b, d, pl.ds(g * LANES, LANES)] = v

        def out_write(cur, b):
            t, b0 = chunk_loc(cur)
            pltpu.async_copy(
                trans_v.at[pl.ds(b, 1)],
                out_hbm.at[pl.ds(t, 1), pl.ds(0, D_MODEL), pl.ds(b0, bw)],
                sem_o[b],
            )

        for p0 in range(nbuf - 1):
            fire(p0, p0)

        upper = n_chunks + nbuf - 1 - ((n_chunks - 1) % nbuf)

        @pl.loop(0, upper, step=nbuf)
        def _outer(s):
            for b in range(nbuf):
                cur = s + b
                bf = (b + nbuf - 1) % nbuf

                @pl.when(jnp.logical_and(cur + nbuf - 1 < n_chunks, cur >= 1))
                def _():
                    drain_out(bf)
                    fire(cur + nbuf - 1, bf)

                @pl.when(jnp.logical_and(cur + nbuf - 1 < n_chunks, cur < 1))
                def _():
                    fire(cur + nbuf - 1, bf)

                @pl.when(cur < n_chunks)
                def _():
                    drain_in(b)
                    transpose(b)
                    out_write(cur, b)

        for b in range(nbuf):
            drain_out(b)

    return body(flat2)


def kernel(x, lut):
    xf = x.reshape(-1).astype(jnp.int32)
    flat = _gather_scale(xf, lut)
    out_t = _transpose_out(flat.reshape(-1, 128))
    return out_t.transpose(2, 0, 1)


# speedup vs baseline: 1.1640x; 1.1640x over previous
import functools
import math

import jax
import jax.numpy as jnp
from jax import lax
from jax.experimental import pallas as pl
from jax.experimental.pallas import tpu as pltpu
from jax.experimental.pallas import tpu_sc as plsc

D_MODEL = 64
LANES = 16
NUM_CORES = 2
NUM_SUBCORES = 16
NUM_WORKERS = NUM_CORES * NUM_SUBCORES
G = 512
K = 1
CHUNK = G * K
NBUF = 3
SCALE = math.sqrt(D_MODEL)


def _sc_embedding(x2d, lut):
    num_groups = x2d.shape[0]
    b_total = num_groups * G
    groups_per_w = num_groups // NUM_WORKERS
    n_chunks = groups_per_w // K

    mesh = plsc.VectorSubcoreMesh(
        core_axis_name="c", subcore_axis_name="s",
        num_cores=NUM_CORES, num_subcores=NUM_SUBCORES,
    )

    @functools.partial(
        pl.kernel,
        out_type=jax.ShapeDtypeStruct((b_total, D_MODEL), jnp.float32),
        mesh=mesh,
        scratch_types=[
            pltpu.VMEM((groups_per_w, G), jnp.int32),
            pltpu.VMEM((NBUF * CHUNK, D_MODEL), jnp.float32),
            [pltpu.SemaphoreType.DMA] * NBUF,
            [pltpu.SemaphoreType.DMA] * NBUF,
        ],
        compiler_params=pltpu.CompilerParams(use_tc_tiling_on_sc=False),
    )
    def body(x_hbm, lut_hbm, out_hbm, idx_all, rows_v, sem_g, sem_o):
        wid = lax.axis_index("s") * NUM_CORES + lax.axis_index("c")
        g0 = wid * groups_per_w

        def rows_at(b):
            return rows_v.at[pl.ds(b * CHUNK, CHUNK)]

        def fire(cur, b):
            for j in range(K):
                pltpu.async_copy(
                    lut_hbm.at[idx_all.at[cur * K + j]],
                    rows_v.at[pl.ds(b * CHUNK + j * G, G)],
                    sem_g[b],
                )

        def drain_gathers(b):
            for j in range(K):
                pltpu.make_async_copy(
                    out_hbm.at[pl.ds(0, G)],
                    rows_v.at[pl.ds(b * CHUNK + j * G, G)],
                    sem_g[b],
                ).wait()

        def drain_out(b):
            pltpu.make_async_copy(
                out_hbm.at[pl.ds(0, CHUNK)], rows_at(b), sem_o[b]
            ).wait()

        def out_write(cur, b):
            pltpu.async_copy(
                rows_at(b),
                out_hbm.at[pl.ds((g0 + cur * K) * G, CHUNK)],
                sem_o[b],
            )

        def scale(b):
            @plsc.parallel_loop(0, CHUNK, unroll=8)
            def _scale(i):
                for j in range(D_MODEL // LANES):
                    sl = pl.ds(j * LANES, LANES)
                    rows_v[b * CHUNK + i, sl] = rows_v[b * CHUNK + i, sl] * SCALE

        pltpu.sync_copy(x_hbm.at[pl.ds(g0, groups_per_w)], idx_all)

        fire(0, 0)
        fire(1, 1)

        n_outer = (n_chunks + NBUF - 1) // NBUF

        @pl.loop(0, n_outer * NBUF, step=NBUF)
        def _outer(s):
            for b in range(NBUF):
                cur = s + b
                bf = (b + 2) % NBUF

                @pl.when(jnp.logical_and(cur + 2 < n_chunks, cur >= 1))
                def _():
                    drain_out(bf)
                    fire(cur + 2, bf)

                @pl.when(jnp.logical_and(cur + 2 < n_chunks, cur < 1))
                def _():
                    fire(cur + 2, bf)

                @pl.when(cur < n_chunks)
                def _():
                    drain_gathers(b)
                    scale(b)
                    out_write(cur, b)

        for b in range(NBUF):
            drain_out(b)

    return body(x2d, lut)


def kernel(x, lut):
    b, t = x.shape
    x2d = x.reshape(-1).astype(jnp.int32).reshape(-1, G)
    out = _sc_embedding(x2d, lut)
    return out.reshape(b, t, D_MODEL)
